# baseline (speedup 1.0000x reference)
import functools

import jax
import jax.numpy as jnp
from jax import lax
from jax.experimental import pallas as pl
from jax.experimental.pallas import tpu as pltpu


def _bce_pos(s):
    return -jnp.maximum(jnp.log(s), -100.0)


def _bce_neg(s):
    return -jnp.maximum(jnp.log(1.0 - s), -100.0)


def _sigmoid(x):
    return 1.0 / (1.0 + jnp.exp(-x))


def _split_dot(val, hot):
    hi = val.astype(jnp.bfloat16).astype(jnp.float32)
    lo = val - hi
    dims = (((0,), (0,)), ((), ()))
    g_hi = lax.dot_general(hi, hot, dims,
                           preferred_element_type=jnp.float32)
    g_lo = lax.dot_general(lo, hot, dims,
                           preferred_element_type=jnp.float32)
    return g_hi + g_lo


def _yolo_kernel(x_ref, t_ref, anch_ref, stride_ref, pred_ref, gath_ref,
                 g4_ref, nsum_ref, loss_ref, hot_ref, *, B, A, G, NO, NT):
    b = pl.program_id(0)
    NC = NO - 5
    cells = G * G
    stride = stride_ref[0, 0]

    timg = t_ref[0:1, :].astype(jnp.int32)
    bx1 = t_ref[2:3, :] / stride
    by1 = t_ref[3:4, :] / stride
    bx2 = t_ref[4:5, :] / stride
    by2 = t_ref[5:6, :] / stride
    bw = bx2 - bx1
    bh = by2 - by1
    cxx = bx1 + bw / 2.0
    cyy = by1 + bh / 2.0
    gx = cxx.astype(jnp.int32)
    gy = cyy.astype(jnp.int32)
    m = gx < 0
    gx = jnp.where(m, 0, gx)
    gy = jnp.where(m, 0, gy)
    m = gy < 0
    gx = jnp.where(m, 0, gx)
    gy = jnp.where(m, 0, gy)
    m = gx >= G
    gx = jnp.where(m, G - 1, gx)
    gy = jnp.where(m, G - 1, gy)
    m = gy >= G
    gx = jnp.where(m, G - 1, gx)
    gy = jnp.where(m, G - 1, gy)
    cell_t = gy * G + gx

    iou_c = []
    for aa in range(A):
        aw = anch_ref[aa, 0] / stride
        ah = anch_ref[aa, 1] / stride
        inter = jnp.minimum(aw, bw) * jnp.minimum(ah, bh)
        iou_c.append(inter / (aw * ah + bw * bh - inter))
    best = jnp.where(iou_c[1] > iou_c[0], 1, 0)
    m01 = jnp.maximum(iou_c[0], iou_c[1])
    best = jnp.where(iou_c[2] > m01, 2, best)

    @pl.when(b == 0)
    def _():
        ri = lax.broadcasted_iota(jnp.int32, (A * cells, NT), 0)
        hot_ref[:] = jnp.where(ri == best * cells + cell_t, 1.0, 0.0)
        gath_ref[:] = jnp.zeros_like(gath_ref)
        g4_ref[:] = jnp.zeros_like(g4_ref)
        nsum_ref[0, 0] = 0.0

    mxp = lax.broadcasted_iota(jnp.int32, (1, G, G), 1).astype(jnp.float32)
    myp = lax.broadcasted_iota(jnp.int32, (1, G, G), 2).astype(jnp.float32)
    nb = jnp.float32(0.0)
    slabs = []
    for aa in range(A):
        slab = x_ref[0, aa]
        s_a = _sigmoid(slab)
        e2 = jnp.exp(slab[2:3])
        e3 = jnp.exp(slab[3:4])
        r0 = (s_a[0:1] + mxp) * stride
        r1 = (s_a[1:2] + myp) * stride
        r2 = (e2 * (anch_ref[aa, 0] / stride)) * stride
        r3 = (e3 * (anch_ref[aa, 1] / stride)) * stride
        out_slab = jnp.concatenate([r0, r1, r2, r3, s_a[4:]], axis=0)
        slabs.append(out_slab)
        nb = nb + jnp.sum(_bce_neg(s_a[4:5]))
    nsum_ref[0, 0] += nb

    pieces = []
    for aa in range(A):
        for w in range(G):
            pieces.append(slabs[aa][:, w, :].T)
    pred_blk = jnp.concatenate(pieces, axis=0)
    pred_ref[0] = pred_blk

    imgm = timg == b
    hot = hot_ref[:]
    contrib = _split_dot(pred_blk, hot)
    gath_ref[:] += jnp.where(imgm, contrib, 0.0)
    cellhot = jnp.zeros((cells, NT), jnp.float32)
    for aa in range(A):
        cellhot = cellhot + hot[aa * cells:(aa + 1) * cells, :]
    g4s = []
    for aa in range(A):
        p4col = pred_blk[aa * cells:(aa + 1) * cells, 4:5]
        g4s.append(_split_dot(p4col, cellhot))
    g4all = jnp.concatenate(g4s, axis=0)
    g4_ref[:] += jnp.where(imgm, g4all, 0.0)

    @pl.when(b == B - 1)
    def _():
        io_t = lax.broadcasted_iota(jnp.int32, (NT, NT), 0)
        io_s = lax.broadcasted_iota(jnp.int32, (NT, NT), 1)
        diag = io_t == io_s

        def _col(row_f32):
            mm = jnp.where(diag, jnp.broadcast_to(row_f32, (NT, NT)), 0.0)
            return jnp.sum(mm, axis=1, keepdims=True)

        tcls = t_ref[1:2, :].astype(jnp.int32)
        key = ((timg * A + best) * cells + cell_t).astype(jnp.float32)
        keq = _col(key) == key
        loser = jnp.any(jnp.logical_and(keq, io_t > io_s),
                        axis=0, keepdims=True)
        w = jnp.where(loser, 0.0, 1.0)
        n_obj = jnp.maximum(jnp.sum(w), 1.0)

        gyf = gy.astype(jnp.float32)
        gxf = gx.astype(jnp.float32)
        anch_tw = jnp.zeros((1, NT), jnp.float32)
        anch_th = jnp.zeros((1, NT), jnp.float32)
        for aa in range(A):
            m_a = best == aa
            anch_tw = anch_tw + jnp.where(m_a, anch_ref[aa, 0] / stride, 0.0)
            anch_th = anch_th + jnp.where(m_a, anch_ref[aa, 1] / stride, 0.0)

        sel = gath_ref[:]
        p0 = sel[0:1, :] / stride - gyf
        p1 = sel[1:2, :] / stride - gxf
        x2r = jnp.log(sel[2:3, :] / (anch_tw * stride))
        x3r = jnp.log(sel[3:4, :] / (anch_th * stride))
        p4 = sel[4:5, :]
        pc = sel[5:, :]

        tx = cxx - jnp.floor(cxx)
        ty = cyy - jnp.floor(cyy)
        twx = jnp.log(bw / anch_tw + 1e-10)
        twy = jnp.log(bh / anch_th + 1e-10)

        lx = jnp.sum(w * (p0 - tx) ** 2)
        ly = jnp.sum(w * (p1 - ty) ** 2)
        lw_ = jnp.sum(w * (x2r - twx) ** 2)
        lh = jnp.sum(w * (x3r - twy) ** 2)
        lobj = jnp.sum(w * _bce_pos(p4))

        cls_i = lax.broadcasted_iota(jnp.int32, (NC, NT), 0)
        oh = jnp.where(cls_i == tcls, 1.0, 0.0)
        bce_c = oh * _bce_pos(pc) + (1.0 - oh) * _bce_neg(pc)
        lcls = jnp.sum(w * bce_c)

        act = []
        key2m = []
        for aa in range(A):
            a_act = jnp.logical_or(best == aa, iou_c[aa] > 0.5)
            k2 = ((timg * A + aa) * cells + cell_t).astype(jnp.float32)
            act.append(a_act)
            key2m.append(jnp.where(a_act, k2, -1.0))
        key2_col = [_col(k) for k in key2m]
        g4v = g4_ref[:]
        corr = jnp.float32(0.0)
        count = jnp.float32(0.0)
        for aa in range(A):
            dup = jnp.zeros((1, NT), jnp.bool_)
            for bb in range(A):
                keq2 = key2_col[bb] == key2m[aa]
                if bb < aa:
                    order = io_t <= io_s
                else:
                    order = io_t < io_s
                dup = jnp.logical_or(
                    dup, jnp.any(jnp.logical_and(keq2, order),
                                 axis=0, keepdims=True))
            fa = jnp.where(jnp.logical_and(act[aa], jnp.logical_not(dup)),
                           1.0, 0.0)
            count = count + jnp.sum(fa)
            corr = corr + jnp.sum(fa * _bce_neg(g4v[aa:aa + 1, :]))
        total = jnp.float32(B * A * cells)
        n_noobj = jnp.maximum(total - count, 1.0)
        lnoobj = (nsum_ref[0, 0] - corr) / n_noobj

        loss = ((lx + ly + lw_ + lh + lobj) / n_obj
                + 100.0 * lnoobj
                + lcls / (n_obj * NC))
        loss_ref[0, 0] = loss


def kernel(x, targets, anchors, img_size):
    B, C, W, H = x.shape
    A = anchors.shape[0]
    NO = C // A
    NT = targets.shape[0]
    G = W
    stride_f = jnp.asarray(img_size // W, jnp.float32).reshape(1, 1)
    x4 = x.reshape(B, A, NO, G, G)
    t_t = targets.T

    out_shapes = (
        jax.ShapeDtypeStruct((B, A * G * G, NO), jnp.float32),
        jax.ShapeDtypeStruct((NO, NT), jnp.float32),
        jax.ShapeDtypeStruct((A, NT), jnp.float32),
        jax.ShapeDtypeStruct((1, 1), jnp.float32),
        jax.ShapeDtypeStruct((1, 1), jnp.float32),
    )
    in_specs = [
        pl.BlockSpec((1, A, NO, G, G), lambda b: (b, 0, 0, 0, 0)),
        pl.BlockSpec((6, NT), lambda b: (0, 0)),
        pl.BlockSpec(memory_space=pltpu.SMEM),
        pl.BlockSpec(memory_space=pltpu.SMEM),
    ]
    out_specs = (
        pl.BlockSpec((1, A * G * G, NO), lambda b: (b, 0, 0)),
        pl.BlockSpec((NO, NT), lambda b: (0, 0)),
        pl.BlockSpec((A, NT), lambda b: (0, 0)),
        pl.BlockSpec(memory_space=pltpu.SMEM),
        pl.BlockSpec(memory_space=pltpu.SMEM),
    )
    body = functools.partial(_yolo_kernel, B=B, A=A, G=G, NO=NO, NT=NT)
    pred, _, _, _, loss = pl.pallas_call(
        body,
        grid=(B,),
        in_specs=in_specs,
        out_specs=out_specs,
        out_shape=out_shapes,
        scratch_shapes=[pltpu.VMEM((A * G * G, NT), jnp.float32)],
    )(x4, t_t, anchors, stride_f)
    return pred, loss[0, 0]


# speedup vs baseline: 1.1777x; 1.1777x over previous
import functools

import jax
import jax.numpy as jnp
from jax import lax
from jax.experimental import pallas as pl
from jax.experimental.pallas import tpu as pltpu


def _bce_pos(s):
    return -jnp.maximum(jnp.log(s), -100.0)


def _bce_neg(s):
    return -jnp.maximum(jnp.log(1.0 - s), -100.0)


def _sigmoid(x):
    return 1.0 / (1.0 + jnp.exp(-x))


def _yolo_kernel(x_ref, t_ref, anch_ref, stride_ref, pred_ref, gath_ref,
                 nsum_ref, loss_ref, hot_ref, *, B, A, G, NO, NT):
    b = pl.program_id(0)
    NC = NO - 5
    cells = G * G
    stride = stride_ref[0, 0]

    timg = t_ref[0:1, :].astype(jnp.int32)
    bx1 = t_ref[2:3, :] / stride
    by1 = t_ref[3:4, :] / stride
    bx2 = t_ref[4:5, :] / stride
    by2 = t_ref[5:6, :] / stride
    bw = bx2 - bx1
    bh = by2 - by1
    cxx = bx1 + bw / 2.0
    cyy = by1 + bh / 2.0
    gx = cxx.astype(jnp.int32)
    gy = cyy.astype(jnp.int32)
    m = gx < 0
    gx = jnp.where(m, 0, gx)
    gy = jnp.where(m, 0, gy)
    m = gy < 0
    gx = jnp.where(m, 0, gx)
    gy = jnp.where(m, 0, gy)
    m = gx >= G
    gx = jnp.where(m, G - 1, gx)
    gy = jnp.where(m, G - 1, gy)
    m = gy >= G
    gx = jnp.where(m, G - 1, gx)
    gy = jnp.where(m, G - 1, gy)
    cell_t = gy * G + gx

    @pl.when(b == 0)
    def _():
        ci = lax.broadcasted_iota(jnp.int32, (cells, NT), 0)
        hot_ref[:] = jnp.where(ci == cell_t, 1.0, 0.0)
        gath_ref[:] = jnp.zeros_like(gath_ref)
        nsum_ref[0, 0] = 0.0

    cell_i = lax.broadcasted_iota(jnp.int32, (1, cells), 1)
    mx = (cell_i // G).astype(jnp.float32)
    my = (cell_i % G).astype(jnp.float32)
    grid01 = jnp.concatenate([mx, my], axis=0)
    nb = jnp.float32(0.0)
    hot = hot_ref[:]
    imgm = timg == b
    for aa in range(A):
        xa = x_ref[0, aa]
        s_a = _sigmoid(xa)
        e2 = jnp.exp(xa[2:3, :])
        e3 = jnp.exp(xa[3:4, :])
        row01 = (s_a[0:2, :] + grid01) * stride
        row2 = (e2 * (anch_ref[aa, 0] / stride)) * stride
        row3 = (e3 * (anch_ref[aa, 1] / stride)) * stride
        blk = jnp.concatenate([row01, row2, row3, s_a[4:, :]], axis=0)
        pred_ref[0, aa] = blk
        nb = nb + jnp.sum(_bce_neg(s_a[4:5, :]))
        contrib = lax.dot_general(
            xa, hot, (((1,), (0,)), ((), ())),
            precision=lax.Precision.DEFAULT,
            preferred_element_type=jnp.float32)
        gath_ref[aa] += jnp.where(imgm, contrib, 0.0)
    nsum_ref[0, 0] += nb

    @pl.when(b == B - 1)
    def _():
        io_t = lax.broadcasted_iota(jnp.int32, (NT, NT), 0)
        io_s = lax.broadcasted_iota(jnp.int32, (NT, NT), 1)
        diag = io_t == io_s

        def _col(row_f32):
            mm = jnp.where(diag, jnp.broadcast_to(row_f32, (NT, NT)), 0.0)
            return jnp.sum(mm, axis=1, keepdims=True)

        tcls = t_ref[1:2, :].astype(jnp.int32)
        iou_c = []
        for aa in range(A):
            aw = anch_ref[aa, 0] / stride
            ah = anch_ref[aa, 1] / stride
            inter = jnp.minimum(aw, bw) * jnp.minimum(ah, bh)
            iou_c.append(inter / (aw * ah + bw * bh - inter))
        best = jnp.where(iou_c[1] > iou_c[0], 1, 0)
        m01 = jnp.maximum(iou_c[0], iou_c[1])
        best = jnp.where(iou_c[2] > m01, 2, best)

        key = ((timg * A + best) * cells + cell_t).astype(jnp.float32)
        keq = _col(key) == key
        loser = jnp.any(jnp.logical_and(keq, io_t > io_s),
                        axis=0, keepdims=True)
        w = jnp.where(loser, 0.0, 1.0)
        n_obj = jnp.maximum(jnp.sum(w), 1.0)

        sel = jnp.zeros((NO, NT), jnp.float32)
        anch_tw = jnp.zeros((1, NT), jnp.float32)
        anch_th = jnp.zeros((1, NT), jnp.float32)
        for aa in range(A):
            m_a = best == aa
            sel = sel + jnp.where(m_a, gath_ref[aa], 0.0)
            anch_tw = anch_tw + jnp.where(m_a, anch_ref[aa, 0] / stride, 0.0)
            anch_th = anch_th + jnp.where(m_a, anch_ref[aa, 1] / stride, 0.0)

        tx = cxx - jnp.floor(cxx)
        ty = cyy - jnp.floor(cyy)
        twx = jnp.log(bw / anch_tw + 1e-10)
        twy = jnp.log(bh / anch_th + 1e-10)

        lx = jnp.sum(w * (_sigmoid(sel[0:1, :]) - tx) ** 2)
        ly = jnp.sum(w * (_sigmoid(sel[1:2, :]) - ty) ** 2)
        lw_ = jnp.sum(w * (sel[2:3, :] - twx) ** 2)
        lh = jnp.sum(w * (sel[3:4, :] - twy) ** 2)
        lobj = jnp.sum(w * _bce_pos(_sigmoid(sel[4:5, :])))

        pc = _sigmoid(sel[5:, :])
        cls_i = lax.broadcasted_iota(jnp.int32, (NC, NT), 0)
        oh = jnp.where(cls_i == tcls, 1.0, 0.0)
        bce_c = oh * _bce_pos(pc) + (1.0 - oh) * _bce_neg(pc)
        lcls = jnp.sum(w * bce_c)

        act = []
        key2m = []
        for aa in range(A):
            a_act = jnp.logical_or(best == aa, iou_c[aa] > 0.5)
            k2 = ((timg * A + aa) * cells + cell_t).astype(jnp.float32)
            act.append(a_act)
            key2m.append(jnp.where(a_act, k2, -1.0))
        key2_col = [_col(k) for k in key2m]
        corr = jnp.float32(0.0)
        count = jnp.float32(0.0)
        for aa in range(A):
            dup = jnp.zeros((1, NT), jnp.bool_)
            for bb in range(A):
                keq2 = key2_col[bb] == key2m[aa]
                if bb < aa:
                    order = io_t <= io_s
                else:
                    order = io_t < io_s
                dup = jnp.logical_or(
                    dup, jnp.any(jnp.logical_and(keq2, order),
                                 axis=0, keepdims=True))
            fa = jnp.where(jnp.logical_and(act[aa], jnp.logical_not(dup)),
                           1.0, 0.0)
            count = count + jnp.sum(fa)
            corr = corr + jnp.sum(fa * _bce_neg(_sigmoid(gath_ref[aa][4:5, :])))
        total = jnp.float32(B * A * cells)
        n_noobj = jnp.maximum(total - count, 1.0)
        lnoobj = (nsum_ref[0, 0] - corr) / n_noobj

        loss = ((lx + ly + lw_ + lh + lobj) / n_obj
                + 100.0 * lnoobj
                + lcls / (n_obj * NC))
        loss_ref[0, 0] = loss


def kernel(x, targets, anchors, img_size):
    B, C, W, H = x.shape
    A = anchors.shape[0]
    NO = C // A
    NT = targets.shape[0]
    G = W
    stride_f = jnp.asarray(img_size // W, jnp.float32).reshape(1, 1)
    x4 = x.reshape(B, A, NO, G * G)
    t_t = targets.T

    grid = (B,)
    out_shapes = (
        jax.ShapeDtypeStruct((B, A, NO, G * G), jnp.float32),
        jax.ShapeDtypeStruct((A, NO, NT), jnp.float32),
        jax.ShapeDtypeStruct((1, 1), jnp.float32),
        jax.ShapeDtypeStruct((1, 1), jnp.float32),
    )
    in_specs = [
        pl.BlockSpec((1, A, NO, G * G), lambda b: (b, 0, 0, 0)),
        pl.BlockSpec((6, NT), lambda b: (0, 0)),
        pl.BlockSpec(memory_space=pltpu.SMEM),
        pl.BlockSpec(memory_space=pltpu.SMEM),
    ]
    out_specs = (
        pl.BlockSpec((1, A, NO, G * G), lambda b: (b, 0, 0, 0)),
        pl.BlockSpec((A, NO, NT), lambda b: (0, 0, 0)),
        pl.BlockSpec(memory_space=pltpu.SMEM),
        pl.BlockSpec(memory_space=pltpu.SMEM),
    )
    body = functools.partial(_yolo_kernel, B=B, A=A, G=G, NO=NO, NT=NT)
    pred4, _, _, loss = pl.pallas_call(
        body,
        grid=grid,
        in_specs=in_specs,
        out_specs=out_specs,
        out_shape=out_shapes,
        scratch_shapes=[pltpu.VMEM((G * G, NT), jnp.float32)],
    )(x4, t_t, anchors, stride_f)
    pred = pred4.transpose(0, 1, 3, 2).reshape(B, A * G * G, NO)
    return pred, loss[0, 0]


# speedup vs baseline: 2.7525x; 2.3373x over previous
import functools

import jax
import jax.numpy as jnp
from jax import lax
from jax.experimental import pallas as pl
from jax.experimental.pallas import tpu as pltpu


def _bce_pos(s):
    return -jnp.maximum(jnp.log(s), -100.0)


def _bce_neg(s):
    return -jnp.maximum(jnp.log(1.0 - s), -100.0)


def _sigmoid(x):
    return 1.0 / (1.0 + jnp.exp(-x))


def _yolo_kernel(x_ref, t_ref, anch_ref, stride_ref, pred_ref, gath_ref,
                 nsum_ref, loss_ref, hot_ref, *, B, A, G, NO, NT):
    b = pl.program_id(0)
    NC = NO - 5
    cells = G * G
    stride = stride_ref[0, 0]

    timg = t_ref[0:1, :].astype(jnp.int32)
    bx1 = t_ref[2:3, :] / stride
    by1 = t_ref[3:4, :] / stride
    bx2 = t_ref[4:5, :] / stride
    by2 = t_ref[5:6, :] / stride
    bw = bx2 - bx1
    bh = by2 - by1
    cxx = bx1 + bw / 2.0
    cyy = by1 + bh / 2.0
    gx = cxx.astype(jnp.int32)
    gy = cyy.astype(jnp.int32)
    m = gx < 0
    gx = jnp.where(m, 0, gx)
    gy = jnp.where(m, 0, gy)
    m = gy < 0
    gx = jnp.where(m, 0, gx)
    gy = jnp.where(m, 0, gy)
    m = gx >= G
    gx = jnp.where(m, G - 1, gx)
    gy = jnp.where(m, G - 1, gy)
    m = gy >= G
    gx = jnp.where(m, G - 1, gx)
    gy = jnp.where(m, G - 1, gy)
    cell_t = gy * G + gx

    @pl.when(b == 0)
    def _():
        ci = lax.broadcasted_iota(jnp.int32, (cells, NT), 0)
        hot_ref[:] = jnp.where(ci == cell_t, 1.0, 0.0)
        gath_ref[:] = jnp.zeros_like(gath_ref)
        nsum_ref[0, 0] = 0.0

    cell_i = lax.broadcasted_iota(jnp.int32, (1, cells), 1)
    mx = (cell_i // G).astype(jnp.float32)
    my = (cell_i % G).astype(jnp.float32)
    grid01 = jnp.concatenate([mx, my], axis=0)
    nb = jnp.float32(0.0)
    hot = hot_ref[:]
    imgm = timg == b
    blks = []
    for aa in range(A):
        xa = x_ref[0, aa * NO:(aa + 1) * NO, :]
        s_a = _sigmoid(xa)
        e2 = jnp.exp(xa[2:3, :])
        e3 = jnp.exp(xa[3:4, :])
        row01 = (s_a[0:2, :] + grid01) * stride
        row2 = (e2 * (anch_ref[aa, 0] / stride)) * stride
        row3 = (e3 * (anch_ref[aa, 1] / stride)) * stride
        blk = jnp.concatenate([row01, row2, row3, s_a[4:, :]], axis=0)
        blks.append(blk.T)
        nb = nb + jnp.sum(_bce_neg(s_a[4:5, :]))
        contrib = lax.dot_general(
            xa, hot, (((1,), (0,)), ((), ())),
            precision=lax.Precision.DEFAULT,
            preferred_element_type=jnp.float32)
        gath_ref[aa] += jnp.where(imgm, contrib, 0.0)
    pred_ref[0] = jnp.concatenate(blks, axis=0)
    nsum_ref[0, 0] += nb

    @pl.when(b == B - 1)
    def _():
        io_t = lax.broadcasted_iota(jnp.int32, (NT, NT), 0)
        io_s = lax.broadcasted_iota(jnp.int32, (NT, NT), 1)
        diag = io_t == io_s

        def _col(row_f32):
            mm = jnp.where(diag, jnp.broadcast_to(row_f32, (NT, NT)), 0.0)
            return jnp.sum(mm, axis=1, keepdims=True)

        tcls = t_ref[1:2, :].astype(jnp.int32)
        iou_c = []
        for aa in range(A):
            aw = anch_ref[aa, 0] / stride
            ah = anch_ref[aa, 1] / stride
            inter = jnp.minimum(aw, bw) * jnp.minimum(ah, bh)
            iou_c.append(inter / (aw * ah + bw * bh - inter))
        best = jnp.where(iou_c[1] > iou_c[0], 1, 0)
        m01 = jnp.maximum(iou_c[0], iou_c[1])
        best = jnp.where(iou_c[2] > m01, 2, best)

        key = ((timg * A + best) * cells + cell_t).astype(jnp.float32)
        keq = _col(key) == key
        loser = jnp.any(jnp.logical_and(keq, io_t > io_s),
                        axis=0, keepdims=True)
        w = jnp.where(loser, 0.0, 1.0)
        n_obj = jnp.maximum(jnp.sum(w), 1.0)

        sel = jnp.zeros((NO, NT), jnp.float32)
        anch_tw = jnp.zeros((1, NT), jnp.float32)
        anch_th = jnp.zeros((1, NT), jnp.float32)
        for aa in range(A):
            m_a = best == aa
            sel = sel + jnp.where(m_a, gath_ref[aa], 0.0)
            anch_tw = anch_tw + jnp.where(m_a, anch_ref[aa, 0] / stride, 0.0)
            anch_th = anch_th + jnp.where(m_a, anch_ref[aa, 1] / stride, 0.0)

        tx = cxx - jnp.floor(cxx)
        ty = cyy - jnp.floor(cyy)
        twx = jnp.log(bw / anch_tw + 1e-10)
        twy = jnp.log(bh / anch_th + 1e-10)

        lx = jnp.sum(w * (_sigmoid(sel[0:1, :]) - tx) ** 2)
        ly = jnp.sum(w * (_sigmoid(sel[1:2, :]) - ty) ** 2)
        lw_ = jnp.sum(w * (sel[2:3, :] - twx) ** 2)
        lh = jnp.sum(w * (sel[3:4, :] - twy) ** 2)
        lobj = jnp.sum(w * _bce_pos(_sigmoid(sel[4:5, :])))

        pc = _sigmoid(sel[5:, :])
        cls_i = lax.broadcasted_iota(jnp.int32, (NC, NT), 0)
        oh = jnp.where(cls_i == tcls, 1.0, 0.0)
        bce_c = oh * _bce_pos(pc) + (1.0 - oh) * _bce_neg(pc)
        lcls = jnp.sum(w * bce_c)

        act = []
        key2m = []
        for aa in range(A):
            a_act = jnp.logical_or(best == aa, iou_c[aa] > 0.5)
            k2 = ((timg * A + aa) * cells + cell_t).astype(jnp.float32)
            act.append(a_act)
            key2m.append(jnp.where(a_act, k2, -1.0))
        key2_col = [_col(k) for k in key2m]
        corr = jnp.float32(0.0)
        count = jnp.float32(0.0)
        for aa in range(A):
            dup = jnp.zeros((1, NT), jnp.bool_)
            for bb in range(A):
                keq2 = key2_col[bb] == key2m[aa]
                if bb < aa:
                    order = io_t <= io_s
                else:
                    order = io_t < io_s
                dup = jnp.logical_or(
                    dup, jnp.any(jnp.logical_and(keq2, order),
                                 axis=0, keepdims=True))
            fa = jnp.where(jnp.logical_and(act[aa], jnp.logical_not(dup)),
                           1.0, 0.0)
            count = count + jnp.sum(fa)
            corr = corr + jnp.sum(fa * _bce_neg(_sigmoid(gath_ref[aa][4:5, :])))
        total = jnp.float32(B * A * cells)
        n_noobj = jnp.maximum(total - count, 1.0)
        lnoobj = (nsum_ref[0, 0] - corr) / n_noobj

        loss = ((lx + ly + lw_ + lh + lobj) / n_obj
                + 100.0 * lnoobj
                + lcls / (n_obj * NC))
        loss_ref[0, 0] = loss


def kernel(x, targets, anchors, img_size):
    B, C, W, H = x.shape
    A = anchors.shape[0]
    NO = C // A
    NT = targets.shape[0]
    G = W
    stride_f = jnp.asarray(img_size // W, jnp.float32).reshape(1, 1)
    x4 = x.reshape(B, C, G * G)
    t_t = targets.T

    grid = (B,)
    out_shapes = (
        jax.ShapeDtypeStruct((B, A * G * G, NO), jnp.float32),
        jax.ShapeDtypeStruct((A, NO, NT), jnp.float32),
        jax.ShapeDtypeStruct((1, 1), jnp.float32),
        jax.ShapeDtypeStruct((1, 1), jnp.float32),
    )
    in_specs = [
        pl.BlockSpec((1, A * NO, G * G), lambda b: (b, 0, 0)),
        pl.BlockSpec((6, NT), lambda b: (0, 0)),
        pl.BlockSpec(memory_space=pltpu.SMEM),
        pl.BlockSpec(memory_space=pltpu.SMEM),
    ]
    out_specs = (
        pl.BlockSpec((1, A * G * G, NO), lambda b: (b, 0, 0)),
        pl.BlockSpec((A, NO, NT), lambda b: (0, 0, 0)),
        pl.BlockSpec(memory_space=pltpu.SMEM),
        pl.BlockSpec(memory_space=pltpu.SMEM),
    )
    body = functools.partial(_yolo_kernel, B=B, A=A, G=G, NO=NO, NT=NT)
    pred, _, _, loss = pl.pallas_call(
        body,
        grid=grid,
        in_specs=in_specs,
        out_specs=out_specs,
        out_shape=out_shapes,
        scratch_shapes=[pltpu.VMEM((G * G, NT), jnp.float32)],
    )(x4, t_t, anchors, stride_f)
    return pred, loss[0, 0]
